# Initial kernel scaffold; baseline (speedup 1.0000x reference)
import jax
import jax.numpy as jnp
from jax.experimental import pallas as pl


def kernel(x, edge_index, W1, b1, W2, b2, W3, b3, W4, b4):
    raise NotImplementedError("write your pallas kernel here")


# speedup vs baseline: 12.0704x; 12.0704x over previous
import functools

import jax
import jax.numpy as jnp
from jax import lax
from jax.experimental import pallas as pl
from jax.experimental.pallas import tpu as pltpu
from jax.experimental.pallas import tpu_sc as plsc

N = 10000
E = 320000
D_IN = 128
H = 64

_NC, _NS = 2, 16
_NW = _NC * _NS
_CB = 128
_K = 80
_EPAD = _NW * _K * _CB
_NACC = 10240
_ZROWS = 64
_RPT = _NACC // _NS

_mesh = plsc.VectorSubcoreMesh(core_axis_name="c", subcore_axis_name="s")


@functools.partial(
    pl.kernel,
    out_type=jax.ShapeDtypeStruct((_NC, _NACC), jnp.float32),
    mesh=_mesh,
    scratch_types=[
        pltpu.VMEM((_K, _CB), jnp.int32),
        pltpu.VMEM((_CB,), jnp.float32),
        pltpu.VMEM((_RPT,), jnp.float32),
        pltpu.VMEM_SHARED((_NACC,), jnp.float32),
    ],
    compiler_params=pltpu.CompilerParams(use_tc_tiling_on_sc=False),
)
def _deg_call(dst_hbm, deg_hbm, dst_v, ones_v, zb_v, dacc):
    cid = lax.axis_index("c")
    sid = lax.axis_index("s")
    wid = cid * _NS + sid

    @pl.loop(0, _RPT // 16)
    def _(i):
        zb_v[pl.ds(i * 16, 16)] = jnp.zeros((16,), jnp.float32)

    @pl.loop(0, _CB // 16)
    def _(i):
        ones_v[pl.ds(i * 16, 16)] = jnp.ones((16,), jnp.float32)

    pltpu.sync_copy(zb_v, dacc.at[pl.ds(sid * _RPT, _RPT)])
    pltpu.sync_copy(dst_hbm.at[pl.ds(wid * _K, _K)], dst_v)
    plsc.subcore_barrier()

    @pl.loop(0, _K)
    def _(k):
        pltpu.sync_copy(ones_v, dacc.at[dst_v.at[k]], add=True)

    plsc.subcore_barrier()
    pltpu.sync_copy(dacc.at[pl.ds(sid * _RPT, _RPT)],
                    deg_hbm.at[cid, pl.ds(sid * _RPT, _RPT)])


@functools.partial(
    pl.kernel,
    out_type=jax.ShapeDtypeStruct((_NC, _NACC, H), jnp.float32),
    mesh=_mesh,
    scratch_types=[
        pltpu.VMEM((_K, _CB), jnp.int32),
        pltpu.VMEM((_K, _CB), jnp.int32),
        pltpu.VMEM((_CB, H), jnp.float32),
        pltpu.VMEM((_CB, H), jnp.float32),
        pltpu.VMEM((_ZROWS, H), jnp.float32),
        pltpu.VMEM_SHARED((_NACC, H), jnp.float32),
        pltpu.SemaphoreType.DMA,
        pltpu.SemaphoreType.DMA,
    ],
    compiler_params=pltpu.CompilerParams(use_tc_tiling_on_sc=False),
)
def _agg_call(hs_hbm, src_hbm, dst_hbm, out_hbm,
              src_v, dst_v, rb0, rb1, zb_v, acc, sem0, sem1):
    cid = lax.axis_index("c")
    sid = lax.axis_index("s")
    wid = cid * _NS + sid

    @pl.loop(0, _ZROWS)
    def _(r):
        for j in range(H // 16):
            zb_v[r, pl.ds(j * 16, 16)] = jnp.zeros((16,), jnp.float32)

    for t in range(_RPT // _ZROWS):
        pltpu.sync_copy(zb_v, acc.at[pl.ds(sid * _RPT + t * _ZROWS, _ZROWS)])

    pltpu.sync_copy(src_hbm.at[pl.ds(wid * _K, _K)], src_v)
    pltpu.sync_copy(dst_hbm.at[pl.ds(wid * _K, _K)], dst_v)
    plsc.subcore_barrier()

    pltpu.async_copy(hs_hbm.at[src_v.at[0]], rb0, sem0)

    @pl.loop(0, _K, step=2)
    def _(k):
        pltpu.make_async_copy(hs_hbm.at[src_v.at[k]], rb0, sem0).wait()
        pltpu.async_copy(hs_hbm.at[src_v.at[k + 1]], rb1, sem1)
        pltpu.sync_copy(rb0, acc.at[dst_v.at[k]], add=True)
        pltpu.make_async_copy(hs_hbm.at[src_v.at[k + 1]], rb1, sem1).wait()

        @pl.when(k + 2 < _K)
        def _():
            pltpu.async_copy(hs_hbm.at[src_v.at[k + 2]], rb0, sem0)

        pltpu.sync_copy(rb1, acc.at[dst_v.at[k + 1]], add=True)

    plsc.subcore_barrier()
    pltpu.sync_copy(acc.at[pl.ds(sid * _RPT, _RPT)],
                    out_hbm.at[cid, pl.ds(sid * _RPT, _RPT)])


_RB = 2000


def _mm_first_body(d0_ref, d1_ref, x_ref, w_ref, hs_ref, dinv_ref):
    di = lax.rsqrt(d0_ref[...] + d1_ref[...] + 1.0)
    dinv_ref[...] = di
    hs_ref[...] = di * jnp.dot(x_ref[...], w_ref[...],
                               preferred_element_type=jnp.float32)


_mm_first = pl.pallas_call(
    _mm_first_body,
    grid=(N // _RB,),
    in_specs=[
        pl.BlockSpec((_RB, 1), lambda i: (i, 0)),
        pl.BlockSpec((_RB, 1), lambda i: (i, 0)),
        pl.BlockSpec((_RB, D_IN), lambda i: (i, 0)),
        pl.BlockSpec((D_IN, H), lambda i: (0, 0)),
    ],
    out_specs=[
        pl.BlockSpec((_RB, H), lambda i: (i, 0)),
        pl.BlockSpec((_RB, 1), lambda i: (i, 0)),
    ],
    out_shape=[
        jax.ShapeDtypeStruct((N, H), jnp.float32),
        jax.ShapeDtypeStruct((N, 1), jnp.float32),
    ],
)


def _mm_mid_body(p0_ref, p1_ref, hsp_ref, dinv_ref, b_ref, w_ref, hs_ref):
    di = dinv_ref[...]
    t = di * (p0_ref[...] + p1_ref[...] + hsp_ref[...]) + b_ref[...]
    a = jnp.maximum(t, 0.0)
    hs_ref[...] = di * jnp.dot(a, w_ref[...],
                               preferred_element_type=jnp.float32)


_mm_mid = pl.pallas_call(
    _mm_mid_body,
    grid=(N // _RB,),
    in_specs=[
        pl.BlockSpec((_RB, H), lambda i: (i, 0)),
        pl.BlockSpec((_RB, H), lambda i: (i, 0)),
        pl.BlockSpec((_RB, H), lambda i: (i, 0)),
        pl.BlockSpec((_RB, 1), lambda i: (i, 0)),
        pl.BlockSpec((1, H), lambda i: (0, 0)),
        pl.BlockSpec((H, H), lambda i: (0, 0)),
    ],
    out_specs=pl.BlockSpec((_RB, H), lambda i: (i, 0)),
    out_shape=jax.ShapeDtypeStruct((N, H), jnp.float32),
)


def _fin_body(p0_ref, p1_ref, hs_ref, dinv_ref, b_ref, out_ref):
    out_ref[...] = (dinv_ref[...] * (p0_ref[...] + p1_ref[...] + hs_ref[...])
                    + b_ref[...])


_fin = pl.pallas_call(
    _fin_body,
    grid=(N // _RB,),
    in_specs=[
        pl.BlockSpec((_RB, H), lambda i: (i, 0)),
        pl.BlockSpec((_RB, H), lambda i: (i, 0)),
        pl.BlockSpec((_RB, H), lambda i: (i, 0)),
        pl.BlockSpec((_RB, 1), lambda i: (i, 0)),
        pl.BlockSpec((1, H), lambda i: (0, 0)),
    ],
    out_specs=pl.BlockSpec((_RB, H), lambda i: (i, 0)),
    out_shape=jax.ShapeDtypeStruct((N, H), jnp.float32),
)


def kernel(x, edge_index, W1, b1, W2, b2, W3, b3, W4, b4):
    src = edge_index[0].astype(jnp.int32)
    dst = edge_index[1].astype(jnp.int32)
    pad = _EPAD - E
    srcp = jnp.concatenate([src, jnp.zeros((pad,), jnp.int32)]).reshape(_NW * _K, _CB)
    dstp = jnp.concatenate([dst, jnp.full((pad,), N, jnp.int32)]).reshape(_NW * _K, _CB)

    deg = _deg_call(dstp)
    d0 = deg[0, :N].reshape(N, 1)
    d1 = deg[1, :N].reshape(N, 1)

    hs1, dinv = _mm_first(d0, d1, x, W1)
    p = _agg_call(hs1, srcp, dstp)
    hs2 = _mm_mid(p[0, :N], p[1, :N], hs1, dinv, b1.reshape(1, H), W2)
    p = _agg_call(hs2, srcp, dstp)
    hs3 = _mm_mid(p[0, :N], p[1, :N], hs2, dinv, b2.reshape(1, H), W3)
    p = _agg_call(hs3, srcp, dstp)
    hs4 = _mm_mid(p[0, :N], p[1, :N], hs3, dinv, b3.reshape(1, H), W4)
    p = _agg_call(hs4, srcp, dstp)
    return _fin(p[0, :N], p[1, :N], hs4, dinv, b4.reshape(1, H))
